# speedup vs baseline: 6.1489x; 6.1489x over previous
import functools

import jax
import jax.numpy as jnp
from jax import lax
from jax.experimental import pallas as pl
from jax.experimental.pallas import tpu as pltpu
from jax.experimental.pallas import tpu_sc as plsc

D = 128
NCH = 3
B = 4096
L = 200
SEGS = NCH * B
NC = 2
NS = 16
NW = NC * NS
SEG_PER_W = SEGS // NW
CHUNKS = 5
K = 40
LANES = D // 16

_mesh = plsc.VectorSubcoreMesh(core_axis_name="c", subcore_axis_name="s")


@functools.partial(
    pl.kernel,
    mesh=_mesh,
    out_type=jax.ShapeDtypeStruct((SEGS, D), jnp.float32),
    scratch_types=[
        pltpu.VMEM((CHUNKS, K), jnp.int32),
        pltpu.VMEM((K, D), jnp.float32),
        pltpu.VMEM((K, D), jnp.float32),
        pltpu.VMEM((1, D), jnp.float32),
        pltpu.SemaphoreType.DMA,
        pltpu.SemaphoreType.DMA,
    ],
)
def _sc_pool(idx_hbm, emb_hbm, out_hbm, idx_v, rows_a, rows_b, acc_v, sem_a, sem_b):
    wid = lax.axis_index("s") * NC + lax.axis_index("c")
    base = wid * SEG_PER_W
    rows = (rows_a, rows_b)
    sems = (sem_a, sem_b)

    def seg_body(i, carry):
        s = base + i
        pltpu.sync_copy(idx_hbm.at[s], idx_v)
        cps = {0: pltpu.async_copy(emb_hbm.at[idx_v.at[0]], rows[0], sems[0])}
        acc = tuple(jnp.zeros((16,), jnp.float32) for _ in range(LANES))
        for j in range(CHUNKS):
            if j + 1 < CHUNKS:
                cps[(j + 1) % 2] = pltpu.async_copy(
                    emb_hbm.at[idx_v.at[j + 1]], rows[(j + 1) % 2],
                    sems[(j + 1) % 2])
            cps[j % 2].wait()
            buf = rows[j % 2]

            def red(r, a):
                return tuple(a[t] + buf[r, pl.ds(16 * t, 16)]
                             for t in range(LANES))

            acc = lax.fori_loop(0, K, red, acc)
        for t in range(LANES):
            acc_v[0, pl.ds(16 * t, 16)] = acc[t] * (1.0 / L)
        pltpu.sync_copy(acc_v, out_hbm.at[pl.ds(s, 1)])
        return carry

    lax.fori_loop(0, SEG_PER_W, seg_body, 0)


BB = 512
H = 256
OPAD = 128


def _mlp_body(p_ref, w1_ref, b1_ref, w2_ref, b2_ref, o_ref):
    p = p_ref[...]
    w1 = w1_ref[...]
    h = jnp.dot(p[0], w1[0:D], preferred_element_type=jnp.float32)
    h = h + jnp.dot(p[1], w1[D:2 * D], preferred_element_type=jnp.float32)
    h = h + jnp.dot(p[2], w1[2 * D:3 * D], preferred_element_type=jnp.float32)
    h = jnp.maximum(h + b1_ref[...], 0.0)
    o_ref[...] = jnp.dot(h, w2_ref[...],
                         preferred_element_type=jnp.float32) + b2_ref[...]


_mlp = pl.pallas_call(
    _mlp_body,
    grid=(B // BB,),
    in_specs=[
        pl.BlockSpec((NCH, BB, D), lambda i: (0, i, 0)),
        pl.BlockSpec((NCH * D, H), lambda i: (0, 0)),
        pl.BlockSpec((1, H), lambda i: (0, 0)),
        pl.BlockSpec((H, OPAD), lambda i: (0, 0)),
        pl.BlockSpec((1, OPAD), lambda i: (0, 0)),
    ],
    out_specs=pl.BlockSpec((BB, OPAD), lambda i: (i, 0)),
    out_shape=jax.ShapeDtypeStruct((B, OPAD), jnp.float32),
)


def kernel(x, emb, fc1_w, fc1_b, fc2_w, fc2_b):
    x = x.astype(jnp.int32)
    idx = jnp.concatenate([x[0], x[2], x[3]], axis=0).reshape(SEGS, CHUNKS, K)
    pooled = _sc_pool(idx, emb)
    pooled3 = pooled.reshape(NCH, B, D)
    w1t = fc1_w.T
    b1 = fc1_b.reshape(1, H)
    w2t = jnp.zeros((H, OPAD), jnp.float32).at[:, :10].set(fc2_w.T)
    b2 = jnp.zeros((1, OPAD), jnp.float32).at[0, :10].set(fc2_b)
    out = _mlp(pooled3, w1t, b1, w2t, b2)
    return out[:, :10]


# speedup vs baseline: 11.1550x; 1.8142x over previous
import functools

import jax
import jax.numpy as jnp
from jax import lax
from jax.experimental import pallas as pl
from jax.experimental.pallas import tpu as pltpu
from jax.experimental.pallas import tpu_sc as plsc

D = 128
NCH = 3
B = 4096
L = 200
SEGS = NCH * B
NC = 2
NS = 16
NW = NC * NS
SEG_PER_W = SEGS // NW
CHUNKS = 2
K = 100
LANES = D // 16
G = 8
NGRP = SEG_PER_W // G

_mesh = plsc.VectorSubcoreMesh(core_axis_name="c", subcore_axis_name="s")


@functools.partial(
    pl.kernel,
    mesh=_mesh,
    out_type=jax.ShapeDtypeStruct((SEGS, D), jnp.float32),
    scratch_types=[
        pltpu.VMEM((G, CHUNKS, K), jnp.int32),
        pltpu.VMEM((K, D), jnp.float32),
        pltpu.VMEM((K, D), jnp.float32),
        pltpu.VMEM((G, D), jnp.float32),
        pltpu.SemaphoreType.DMA,
        pltpu.SemaphoreType.DMA,
    ],
)
def _sc_pool(idx_hbm, emb_hbm, out_hbm, idx_v, rows_a, rows_b, ostage, sem_a, sem_b):
    wid = lax.axis_index("s") * NC + lax.axis_index("c")
    base = wid * SEG_PER_W
    rows = (rows_a, rows_b)
    sems = (sem_a, sem_b)
    NCK = G * CHUNKS

    def grp_body(g, carry):
        s0 = base + g * G
        pltpu.sync_copy(idx_hbm.at[pl.ds(s0, G)], idx_v)
        cps = {0: pltpu.async_copy(emb_hbm.at[idx_v.at[0, 0]], rows[0], sems[0])}
        acc = None
        for t in range(NCK):
            seg, j = divmod(t, CHUNKS)
            if t + 1 < NCK:
                seg2, j2 = divmod(t + 1, CHUNKS)
                cps[(t + 1) % 2] = pltpu.async_copy(
                    emb_hbm.at[idx_v.at[seg2, j2]], rows[(t + 1) % 2],
                    sems[(t + 1) % 2])
            cps[t % 2].wait()
            buf = rows[t % 2]
            if j == 0:
                acc = tuple(jnp.zeros((16,), jnp.float32) for _ in range(LANES))

            def red(r, a):
                return tuple(a[tt] + buf[2 * r, pl.ds(16 * tt, 16)]
                             + buf[2 * r + 1, pl.ds(16 * tt, 16)]
                             for tt in range(LANES))

            acc = lax.fori_loop(0, K // 2, red, acc)
            if j == CHUNKS - 1:
                for tt in range(LANES):
                    ostage[seg, pl.ds(16 * tt, 16)] = acc[tt] * (1.0 / L)
        pltpu.sync_copy(ostage, out_hbm.at[pl.ds(s0, G)])
        return carry

    lax.fori_loop(0, NGRP, grp_body, 0)


BB = 512
H = 256
OPAD = 128


def _mlp_body(p_ref, w1_ref, b1_ref, w2_ref, b2_ref, o_ref):
    p = p_ref[...]
    w1 = w1_ref[...]
    h = jnp.dot(p[0], w1[0:D], preferred_element_type=jnp.float32)
    h = h + jnp.dot(p[1], w1[D:2 * D], preferred_element_type=jnp.float32)
    h = h + jnp.dot(p[2], w1[2 * D:3 * D], preferred_element_type=jnp.float32)
    h = jnp.maximum(h + b1_ref[...], 0.0)
    o_ref[...] = jnp.dot(h, w2_ref[...],
                         preferred_element_type=jnp.float32) + b2_ref[...]


_mlp = pl.pallas_call(
    _mlp_body,
    grid=(B // BB,),
    in_specs=[
        pl.BlockSpec((NCH, BB, D), lambda i: (0, i, 0)),
        pl.BlockSpec((NCH * D, H), lambda i: (0, 0)),
        pl.BlockSpec((1, H), lambda i: (0, 0)),
        pl.BlockSpec((H, OPAD), lambda i: (0, 0)),
        pl.BlockSpec((1, OPAD), lambda i: (0, 0)),
    ],
    out_specs=pl.BlockSpec((BB, OPAD), lambda i: (i, 0)),
    out_shape=jax.ShapeDtypeStruct((B, OPAD), jnp.float32),
)


def kernel(x, emb, fc1_w, fc1_b, fc2_w, fc2_b):
    x = x.astype(jnp.int32)
    idx = jnp.concatenate([x[0], x[2], x[3]], axis=0).reshape(SEGS, CHUNKS, K)
    pooled = _sc_pool(idx, emb)
    pooled3 = pooled.reshape(NCH, B, D)
    w1t = fc1_w.T
    b1 = fc1_b.reshape(1, H)
    w2t = jnp.zeros((H, OPAD), jnp.float32).at[:, :10].set(fc2_w.T)
    b2 = jnp.zeros((1, OPAD), jnp.float32).at[0, :10].set(fc2_b)
    out = _mlp(pooled3, w1t, b1, w2t, b2)
    return out[:, :10]
